# baseline (speedup 1.0000x reference)
import functools

import jax
import jax.numpy as jnp
from jax import lax
from jax.experimental import pallas as pl
from jax.experimental.pallas import tpu as pltpu
from jax.experimental.pallas import tpu_sc as plsc

N = 8192
D = 512
E = 131072
CW = 256
NCHUNK = D // CW
NC = 2
NS = 16
NW = NC * NS
RPT = N // NW
ECH = 4096
NSC = E // ECH
GB = 64
CNTW = 16


def _sc_segment_sum(x0, x1, e3):
    mesh = plsc.VectorSubcoreMesh(core_axis_name="c", subcore_axis_name="s")

    @functools.partial(
        pl.kernel,
        mesh=mesh,
        out_type=[
            jax.ShapeDtypeStruct((NCHUNK, N, CW), jnp.float32),
            jax.ShapeDtypeStruct((N, CNTW), jnp.float32),
        ],
        scratch_types=[
            pltpu.VMEM((2, ECH), jnp.int32),
            pltpu.VMEM((ECH + GB,), jnp.int32),
            pltpu.VMEM((GB,), jnp.int32),
            pltpu.VMEM((GB, CW), jnp.float32),
            pltpu.VMEM((RPT, CW), jnp.float32),
            pltpu.VMEM((RPT, CNTW), jnp.float32),
        ],
    )
    def seg_kernel(x0_h, x1_h, e_h, agg_out, cnt_out,
                   ebuf, mpk, gidx, gbuf, slab, cnt):
        cid = lax.axis_index("c")
        sid = lax.axis_index("s")
        wid = sid * NC + cid
        lo = wid * RPT

        zvec = jnp.zeros((16,), jnp.float32)
        ovec = jnp.ones((16,), jnp.float32)
        xs = (x0_h, x1_h)

        for c in range(NCHUNK):
            xc = xs[c]

            @plsc.parallel_loop(0, RPT)
            def fill_zero(i):
                for k in range(CW // 16):
                    slab[i, pl.ds(k * 16, 16)] = zvec
                if c == 0:
                    cnt[i, pl.ds(0, 16)] = zvec

            def scan_chunk(kc, carry0):
                pltpu.sync_copy(e_h.at[kc], ebuf)

                def match(v, off):
                    svec = ebuf[0, pl.ds(v * 16, 16)]
                    dvec = ebuf[1, pl.ds(v * 16, 16)]
                    lvec = dvec - lo
                    mbit = ((lvec | (RPT - 1 - lvec)) >> 31) + 1
                    pvec = (svec << 8) + lvec

                    def append(off2):
                        o = off2
                        for l in range(16):
                            mpk[pl.ds(o, 16)] = jnp.broadcast_to(
                                pvec[l], (16,))
                            o = o + mbit[l]
                        return o

                    return append(off)

                nm = lax.fori_loop(0, ECH // 16, match, 0)

                for t in range(GB // 16):
                    mpk[pl.ds(nm + t * 16, 16)] = jnp.zeros((16,), jnp.int32)

                def gather_batch(b, carry1):
                    for t in range(GB // 16):
                        gidx[pl.ds(t * 16, 16)] = (
                            mpk[pl.ds(b * GB + t * 16, 16)] >> 8)
                    pltpu.sync_copy(xc.at[gidx], gbuf)
                    mb = jnp.minimum(nm - b * GB, GB)

                    @plsc.parallel_loop(0, mb, unroll=2)
                    def accum(i):
                        pk = mpk[pl.ds(b * GB + i, 16)][0]
                        r = pk & (RPT - 1)
                        for k in range(CW // 16):
                            plsc.addupdate(
                                slab.at[r, pl.ds(k * 16, 16)],
                                gbuf[i, pl.ds(k * 16, 16)])
                        if c == 0:
                            plsc.addupdate(cnt.at[r, pl.ds(0, 16)], ovec)
                    return carry1

                lax.fori_loop(0, (nm + GB - 1) // GB, gather_batch, 0)
                return carry0

            lax.fori_loop(0, NSC, scan_chunk, 0)

            pltpu.sync_copy(slab, agg_out.at[c, pl.ds(lo, RPT)])
            if c == 0:
                pltpu.sync_copy(cnt, cnt_out.at[pl.ds(lo, RPT)])

    return seg_kernel(x0, x1, e3)


BM = 1024


def _mlp(agg_part, cnt_part, x, W_l, b_l, W_r, Wa, ba, W1, b1, W2, b2, W3p, b3p):
    def body(aggp_r, cntp_r, x_r, wl_r, bl_r, wr_r, wa_r, ba_r,
             w1_r, b1_r, w2_r, b2_r, w3_r, b3_r, z_r):
        ap = aggp_r[...]
        cnt = cntp_r[...][:, 0]
        inv = 1.0 / jnp.maximum(cnt, 1.0)
        agg = jnp.concatenate([ap[cc] for cc in range(NCHUNK)], axis=1)
        agg = agg * inv[:, None]
        h = jnp.dot(agg, wl_r[...], preferred_element_type=jnp.float32)
        h = h + jnp.dot(x_r[...], wr_r[...], preferred_element_type=jnp.float32)
        h = jnp.maximum(h + bl_r[...], 0.0)
        h = jnp.maximum(
            jnp.dot(h, wa_r[...], preferred_element_type=jnp.float32)
            + ba_r[...], 0.0)
        h = jnp.maximum(
            jnp.dot(h, w1_r[...], preferred_element_type=jnp.float32)
            + b1_r[...], 0.0)
        h = jnp.maximum(
            jnp.dot(h, w2_r[...], preferred_element_type=jnp.float32)
            + b2_r[...], 0.0)
        z_r[...] = (jnp.dot(h, w3_r[...], preferred_element_type=jnp.float32)
                    + b3_r[...])

    full = lambda a: pl.BlockSpec(a.shape, lambda i: (0,) * a.ndim)
    return pl.pallas_call(
        body,
        grid=(N // BM,),
        in_specs=[
            pl.BlockSpec((NCHUNK, BM, CW), lambda i: (0, i, 0)),
            pl.BlockSpec((BM, CNTW), lambda i: (i, 0)),
            pl.BlockSpec((BM, D), lambda i: (i, 0)),
            full(W_l), full(b_l), full(W_r), full(Wa), full(ba),
            full(W1), full(b1), full(W2), full(b2), full(W3p), full(b3p),
        ],
        out_specs=pl.BlockSpec((BM, 128), lambda i: (i, 0)),
        out_shape=jax.ShapeDtypeStruct((N, 128), jnp.float32),
    )(agg_part, cnt_part, x, W_l, b_l, W_r, Wa, ba, W1, b1, W2, b2, W3p, b3p)


CM = 1024
CN = 1024


def _cdist(z):
    def body(zi_r, zj_r, out_r):
        zi = zi_r[...]
        zj = zj_r[...]
        g = lax.dot_general(zi, zj, (((1,), (1,)), ((), ())),
                            preferred_element_type=jnp.float32)
        sqi = jnp.sum(zi * zi, axis=1)[:, None]
        sqj = jnp.sum(zj * zj, axis=1)[None, :]
        d2 = sqi + sqj - 2.0 * g
        out_r[...] = jnp.sqrt(jnp.maximum(d2, 1e-12))

    return pl.pallas_call(
        body,
        grid=(N // CM, N // CN),
        in_specs=[
            pl.BlockSpec((CM, 128), lambda i, j: (i, 0)),
            pl.BlockSpec((CN, 128), lambda i, j: (j, 0)),
        ],
        out_specs=pl.BlockSpec((CM, CN), lambda i, j: (i, j)),
        out_shape=jax.ShapeDtypeStruct((N, N), jnp.float32),
    )(z, z)


def kernel(x, edge_index, W_l, b_l, W_r, Wa, ba, W1, b1, W2, b2, W3, b3):
    e3 = jnp.stack([edge_index[0].reshape(NSC, ECH),
                    edge_index[1].reshape(NSC, ECH)], axis=1)
    x0 = x[:, :CW]
    x1 = x[:, CW:]
    agg_part, cnt_part = _sc_segment_sum(x0, x1, e3)
    W3p = jnp.pad(W3, ((0, 0), (0, 128 - 3)))
    b3p = jnp.pad(b3, (0, 128 - 3)).reshape(1, 128)
    z = _mlp(agg_part, cnt_part, x,
             W_l, b_l.reshape(1, D), W_r,
             Wa, ba.reshape(1, 256),
             W1, b1.reshape(1, 128),
             W2, b2.reshape(1, 64),
             W3p, b3p)
    return _cdist(z)


# speedup vs baseline: 1.0013x; 1.0013x over previous
import functools

import jax
import jax.numpy as jnp
from jax import lax
from jax.experimental import pallas as pl
from jax.experimental.pallas import tpu as pltpu
from jax.experimental.pallas import tpu_sc as plsc

N = 8192
D = 512
E = 131072
CW = 256
NCHUNK = D // CW
NC = 2
NS = 16
NW = NC * NS
RPT = N // NW
ECH = 4096
NSC = E // ECH
GB = 64
CNTW = 16


def _sc_segment_sum(x0, x1, e3):
    mesh = plsc.VectorSubcoreMesh(core_axis_name="c", subcore_axis_name="s")

    @functools.partial(
        pl.kernel,
        mesh=mesh,
        out_type=[
            jax.ShapeDtypeStruct((NCHUNK, N, CW), jnp.float32),
            jax.ShapeDtypeStruct((N, CNTW), jnp.float32),
        ],
        scratch_types=[
            pltpu.VMEM((2, ECH), jnp.int32),
            pltpu.VMEM((ECH + GB,), jnp.int32),
            pltpu.VMEM((GB,), jnp.int32),
            pltpu.VMEM((GB, CW), jnp.float32),
            pltpu.VMEM((RPT, CW), jnp.float32),
            pltpu.VMEM((RPT, CNTW), jnp.float32),
        ],
    )
    def seg_kernel(x0_h, x1_h, e_h, agg_out, cnt_out,
                   ebuf, mpk, gidx, gbuf, slab, cnt):
        cid = lax.axis_index("c")
        sid = lax.axis_index("s")
        wid = sid * NC + cid
        lo = wid * RPT

        zvec = jnp.zeros((16,), jnp.float32)
        ovec = jnp.ones((16,), jnp.float32)
        xs = (x0_h, x1_h)

        for c in range(NCHUNK):
            xc = xs[c]

            @plsc.parallel_loop(0, RPT)
            def fill_zero(i):
                for k in range(CW // 16):
                    slab[i, pl.ds(k * 16, 16)] = zvec
                if c == 0:
                    cnt[i, pl.ds(0, 16)] = zvec

            def scan_chunk(kc, carry0):
                pltpu.sync_copy(e_h.at[kc], ebuf)

                def match(v, off):
                    svec = ebuf[0, pl.ds(v * 16, 16)]
                    dvec = ebuf[1, pl.ds(v * 16, 16)]
                    lvec = dvec - lo
                    mbit = ((lvec | (RPT - 1 - lvec)) >> 31) + 1
                    pvec = (svec << 8) + lvec

                    def append(off2):
                        o = off2
                        for l in range(16):
                            mpk[pl.ds(o, 16)] = jnp.broadcast_to(
                                pvec[l], (16,))
                            o = o + mbit[l]
                        return o

                    return append(off)

                nm = lax.fori_loop(0, ECH // 16, match, 0)

                for t in range(GB // 16):
                    mpk[pl.ds(nm + t * 16, 16)] = jnp.zeros((16,), jnp.int32)

                def gather_batch(b, carry1):
                    for t in range(GB // 16):
                        gidx[pl.ds(t * 16, 16)] = (
                            mpk[pl.ds(b * GB + t * 16, 16)] >> 8)
                    pltpu.sync_copy(xc.at[gidx], gbuf)
                    mb = jnp.minimum(nm - b * GB, GB)

                    @plsc.parallel_loop(0, mb)
                    def accum(i):
                        pk = mpk[pl.ds(b * GB + i, 16)][0]
                        r = pk & (RPT - 1)
                        for k in range(CW // 16):
                            plsc.addupdate(
                                slab.at[r, pl.ds(k * 16, 16)],
                                gbuf[i, pl.ds(k * 16, 16)])
                        if c == 0:
                            plsc.addupdate(cnt.at[r, pl.ds(0, 16)], ovec)
                    return carry1

                lax.fori_loop(0, (nm + GB - 1) // GB, gather_batch, 0)
                return carry0

            lax.fori_loop(0, NSC, scan_chunk, 0)

            pltpu.sync_copy(slab, agg_out.at[c, pl.ds(lo, RPT)])
            if c == 0:
                pltpu.sync_copy(cnt, cnt_out.at[pl.ds(lo, RPT)])

    return seg_kernel(x0, x1, e3)


BM = 1024


def _mlp(agg_part, cnt_part, x, W_l, b_l, W_r, Wa, ba, W1, b1, W2, b2, W3p, b3p):
    def body(aggp_r, cntp_r, x_r, wl_r, bl_r, wr_r, wa_r, ba_r,
             w1_r, b1_r, w2_r, b2_r, w3_r, b3_r, z_r):
        ap = aggp_r[...]
        cnt = cntp_r[...][:, 0]
        inv = 1.0 / jnp.maximum(cnt, 1.0)
        agg = jnp.concatenate([ap[cc] for cc in range(NCHUNK)], axis=1)
        agg = agg * inv[:, None]
        h = jnp.dot(agg, wl_r[...], preferred_element_type=jnp.float32)
        h = h + jnp.dot(x_r[...], wr_r[...], preferred_element_type=jnp.float32)
        h = jnp.maximum(h + bl_r[...], 0.0)
        h = jnp.maximum(
            jnp.dot(h, wa_r[...], preferred_element_type=jnp.float32)
            + ba_r[...], 0.0)
        h = jnp.maximum(
            jnp.dot(h, w1_r[...], preferred_element_type=jnp.float32)
            + b1_r[...], 0.0)
        h = jnp.maximum(
            jnp.dot(h, w2_r[...], preferred_element_type=jnp.float32)
            + b2_r[...], 0.0)
        z_r[...] = (jnp.dot(h, w3_r[...], preferred_element_type=jnp.float32)
                    + b3_r[...])

    full = lambda a: pl.BlockSpec(a.shape, lambda i: (0,) * a.ndim)
    return pl.pallas_call(
        body,
        grid=(N // BM,),
        in_specs=[
            pl.BlockSpec((NCHUNK, BM, CW), lambda i: (0, i, 0)),
            pl.BlockSpec((BM, CNTW), lambda i: (i, 0)),
            pl.BlockSpec((BM, D), lambda i: (i, 0)),
            full(W_l), full(b_l), full(W_r), full(Wa), full(ba),
            full(W1), full(b1), full(W2), full(b2), full(W3p), full(b3p),
        ],
        out_specs=pl.BlockSpec((BM, 128), lambda i: (i, 0)),
        out_shape=jax.ShapeDtypeStruct((N, 128), jnp.float32),
    )(agg_part, cnt_part, x, W_l, b_l, W_r, Wa, ba, W1, b1, W2, b2, W3p, b3p)


CM = 1024
CN = 1024


def _cdist(z):
    def body(zi_r, zj_r, out_r):
        zi = zi_r[...]
        zj = zj_r[...]
        g = lax.dot_general(zi, zj, (((1,), (1,)), ((), ())),
                            preferred_element_type=jnp.float32)
        sqi = jnp.sum(zi * zi, axis=1)[:, None]
        sqj = jnp.sum(zj * zj, axis=1)[None, :]
        d2 = sqi + sqj - 2.0 * g
        out_r[...] = jnp.sqrt(jnp.maximum(d2, 1e-12))

    return pl.pallas_call(
        body,
        grid=(N // CM, N // CN),
        in_specs=[
            pl.BlockSpec((CM, 128), lambda i, j: (i, 0)),
            pl.BlockSpec((CN, 128), lambda i, j: (j, 0)),
        ],
        out_specs=pl.BlockSpec((CM, CN), lambda i, j: (i, j)),
        out_shape=jax.ShapeDtypeStruct((N, N), jnp.float32),
    )(z, z)


def kernel(x, edge_index, W_l, b_l, W_r, Wa, ba, W1, b1, W2, b2, W3, b3):
    e3 = jnp.stack([edge_index[0].reshape(NSC, ECH),
                    edge_index[1].reshape(NSC, ECH)], axis=1)
    x0 = x[:, :CW]
    x1 = x[:, CW:]
    agg_part, cnt_part = _sc_segment_sum(x0, x1, e3)
    W3p = jnp.pad(W3, ((0, 0), (0, 128 - 3)))
    b3p = jnp.pad(b3, (0, 128 - 3)).reshape(1, 128)
    z = _mlp(agg_part, cnt_part, x,
             W_l, b_l.reshape(1, D), W_r,
             Wa, ba.reshape(1, 256),
             W1, b1.reshape(1, 128),
             W2, b2.reshape(1, 64),
             W3p, b3p)
    return _cdist(z)
